# baseline (speedup 1.0000x reference)
import functools

import jax
import jax.numpy as jnp
from jax import lax
from jax.experimental import pallas as pl
from jax.experimental.pallas import tpu as pltpu
from jax.experimental.pallas import tpu_sc as plsc

N = 10000
DIN = 256
DOUT = 256
H = 8
E = 160000

NP = 10112
TILES = 32
EPT = 5120
EP = TILES * EPT
BLK = 64
NBLK = EPT // BLK
STRIPE = NP // 16
NCHUNK = 16


def _dense_body(x_ref, w_ref, a1_ref, a2_ref, h_ref, st1_ref, st2_ref):
    x = x_ref[...]
    s1l, s2l = [], []
    for i in range(H):
        hi = jnp.dot(x, w_ref[i], preferred_element_type=jnp.float32)
        h_ref[:, DOUT * i:DOUT * (i + 1)] = hi
        s1l.append(jnp.dot(hi, a1_ref[i], preferred_element_type=jnp.float32))
        s2l.append(jnp.dot(hi, a2_ref[i], preferred_element_type=jnp.float32))
    zpad = jnp.zeros((x.shape[0], 128 - 2 * H), jnp.float32)
    st1_ref[...] = jnp.concatenate(s1l + s2l + [zpad], axis=1)
    st2_ref[...] = jnp.concatenate(s2l + s1l + [zpad], axis=1)


def _dense_stage(x, W, a1, a2):
    RB = 1000
    grid = (N // RB,)
    return pl.pallas_call(
        _dense_body,
        grid=grid,
        in_specs=[
            pl.BlockSpec((RB, DIN), lambda r: (r, 0)),
            pl.BlockSpec((H, DIN, DOUT), lambda r: (0, 0, 0)),
            pl.BlockSpec((H, DOUT, 1), lambda r: (0, 0, 0)),
            pl.BlockSpec((H, DOUT, 1), lambda r: (0, 0, 0)),
        ],
        out_specs=[
            pl.BlockSpec((RB, H * DOUT), lambda r: (r, 0)),
            pl.BlockSpec((RB, 128), lambda r: (r, 0)),
            pl.BlockSpec((RB, 128), lambda r: (r, 0)),
        ],
        out_shape=[
            jax.ShapeDtypeStruct((N, H * DOUT), jnp.float32),
            jax.ShapeDtypeStruct((N, 128), jnp.float32),
            jax.ShapeDtypeStruct((N, 128), jnp.float32),
        ],
    )(x, W, a1, a2)


def _sc_body(ht, s1t, s2t, rw, cl, z128, uout, dout, exout,
             row_v, col_v, g1_v, g2_v, xb_v, xb2_v, ix_v, ix2_v,
             iosem0, iosem1, ssem0, ssem1, acc):
    cid = lax.axis_index("c")
    tid = lax.axis_index("s")
    wid = tid * 2 + cid

    pltpu.sync_copy(rw.at[wid], row_v)
    pltpu.sync_copy(cl.at[wid], col_v)

    lane = lax.broadcasted_iota(jnp.int32, (16,), 0)
    m8 = lane < 8

    pltpu.sync_copy(z128.at[pl.ds(tid * STRIPE, STRIPE)],
                    acc.at[pl.ds(tid * STRIPE, STRIPE)])
    plsc.subcore_barrier()

    def blk_a(b, carry):
        pltpu.sync_copy(s1t.at[row_v.at[b]], g1_v)
        pltpu.sync_copy(s2t.at[col_v.at[b]], g2_v)

        @plsc.parallel_loop(0, BLK, 1, unroll=4)
        def edge_a(j):
            s = g1_v[j, pl.ds(0, 16)] + g2_v[j, pl.ds(0, 16)]
            e = jnp.where(s > 0.0, s, 0.2 * s)
            ex = jnp.where(m8, jnp.exp(e), 0.0)
            g2_v[j, pl.ds(0, 16)] = ex
            xb_v[pl.ds(j * 16, 16)] = ex
        pltpu.sync_copy(g2_v, acc.at[row_v.at[b]], add=True)
        pltpu.sync_copy(
            xb_v, exout.at[pl.ds((wid * EPT + b * BLK) * 16, BLK * 16)])
        return carry

    lax.fori_loop(0, NBLK, blk_a, 0)
    plsc.subcore_barrier()
    pltpu.sync_copy(acc.at[pl.ds(tid * STRIPE, STRIPE)],
                    dout.at[pl.ds(cid * NP + tid * STRIPE, STRIPE)])
    plsc.subcore_barrier()

    bufs = (g1_v, g2_v)
    xbs = (xb_v, xb2_v)
    ixs = (ix_v, ix2_v)
    iosems = (iosem0, iosem1)
    ssems = (ssem0, ssem1)

    def chunk(kk, carry):
        pltpu.sync_copy(z128.at[pl.ds(tid * STRIPE, STRIPE)],
                        acc.at[pl.ds(tid * STRIPE, STRIPE)])
        plsc.subcore_barrier()
        hd16 = jnp.full((16,), kk // 2, jnp.int32)

        def issue_in(b, p):
            for l in range(BLK // 16):
                ixs[p][pl.ds(l * 16, 16)] = (
                    col_v[b, pl.ds(l * 16, 16)] * 16 + kk)
            pltpu.async_copy(ht.at[ixs[p]], bufs[p], iosems[p])
            pltpu.async_copy(
                exout.at[pl.ds((wid * EPT + b * BLK) * 16, BLK * 16)],
                xbs[p], iosems[p])

        issue_in(0, 0)

        def pair(b2, c1):
            for p in (0, 1):
                b = b2 * 2 + p
                pltpu.make_async_copy(ht.at[ixs[p]], bufs[p],
                                      iosems[p]).wait()
                pltpu.make_async_copy(
                    exout.at[pl.ds((wid * EPT + b * BLK) * 16, BLK * 16)],
                    xbs[p], iosems[p]).wait()

                @plsc.parallel_loop(0, BLK, 1, unroll=4)
                def edge_c(j, _p=p):
                    exrow = xbs[_p][pl.ds(j * 16, 16)]
                    wv = lax.gather(
                        exrow, hd16[:, None],
                        lax.GatherDimensionNumbers(
                            offset_dims=(), collapsed_slice_dims=(0,),
                            start_index_map=(0,)),
                        slice_sizes=(1,),
                        mode=lax.GatherScatterMode.PROMISE_IN_BOUNDS)
                    for t in range(8):
                        bufs[_p][j, pl.ds(t * 16, 16)] = (
                            bufs[_p][j, pl.ds(t * 16, 16)] * wv)
                pltpu.async_copy(bufs[p], acc.at[row_v.at[b]], ssems[p],
                                 add=True)

                if p == 0:
                    @pl.when(b2 > 0)
                    def _wait_prev():
                        pltpu.make_async_copy(
                            bufs[1], acc.at[row_v.at[b - 1]],
                            ssems[1]).wait()
                    issue_in(b + 1, 1)
                else:
                    pltpu.make_async_copy(
                        bufs[0], acc.at[row_v.at[b - 1]], ssems[0]).wait()

                    @pl.when(b2 < NBLK // 2 - 1)
                    def _issue_next():
                        issue_in(b + 1, 0)
            return c1

        lax.fori_loop(0, NBLK // 2, pair, 0)
        pltpu.make_async_copy(bufs[1], acc.at[row_v.at[NBLK - 1]],
                              ssems[1]).wait()
        plsc.subcore_barrier()
        pltpu.sync_copy(
            acc.at[pl.ds(tid * STRIPE, STRIPE)],
            uout.at[pl.ds((cid * NCHUNK + kk) * NP + tid * STRIPE, STRIPE)])
        plsc.subcore_barrier()
        return carry

    lax.fori_loop(0, NCHUNK, chunk, 0)


def _sc_stage(htable, st1, st2, row3, col3, z128):
    mesh = plsc.VectorSubcoreMesh(core_axis_name="c", subcore_axis_name="s")
    kern = functools.partial(
        pl.kernel,
        out_type=[
            jax.ShapeDtypeStruct((2 * NCHUNK * NP, 128), jnp.float32),
            jax.ShapeDtypeStruct((2 * NP, 128), jnp.float32),
            jax.ShapeDtypeStruct((EP * 16,), jnp.float32),
        ],
        mesh=mesh,
        scratch_types=[
            pltpu.VMEM((NBLK, BLK), jnp.int32),
            pltpu.VMEM((NBLK, BLK), jnp.int32),
            pltpu.VMEM((BLK, 128), jnp.float32),
            pltpu.VMEM((BLK, 128), jnp.float32),
            pltpu.VMEM((BLK * 16,), jnp.float32),
            pltpu.VMEM((BLK * 16,), jnp.float32),
            pltpu.VMEM((BLK,), jnp.int32),
            pltpu.VMEM((BLK,), jnp.int32),
            pltpu.SemaphoreType.DMA,
            pltpu.SemaphoreType.DMA,
            pltpu.SemaphoreType.DMA,
            pltpu.SemaphoreType.DMA,
            pltpu.VMEM_SHARED((NP, 128), jnp.float32),
        ],
    )(_sc_body)
    return kern(htable, st1, st2, row3, col3, z128)


def _epi_body(u_ref, d_ref, b_ref, o_ref):
    dsum = d_ref[0] + d_ref[1]
    for k in range(NCHUNK):
        i, c = k // 2, k % 2
        u = u_ref[0, k] + u_ref[1, k]
        dk = dsum[:, i:i + 1]
        val = jnp.where(dk > 0.0, u / dk, 0.0)
        val = val + b_ref[i, 128 * c:128 * (c + 1)][None, :]
        o_ref[:, 128 * k:128 * (k + 1)] = jnp.maximum(val, 0.0)


def _epilogue(uout, den, bias):
    RB = 1000
    grid = (N // RB,)
    return pl.pallas_call(
        _epi_body,
        grid=grid,
        in_specs=[
            pl.BlockSpec((2, NCHUNK, RB, 128), lambda r: (0, 0, r, 0)),
            pl.BlockSpec((2, RB, 128), lambda r: (0, r, 0)),
            pl.BlockSpec((H, 2 * 128), lambda r: (0, 0)),
        ],
        out_specs=pl.BlockSpec((RB, H * DOUT), lambda r: (r, 0)),
        out_shape=jax.ShapeDtypeStruct((N, H * DOUT), jnp.float32),
    )(uout, den, bias)


def kernel(x, edge_index, W, a1, a2, bias):
    h, st1, st2 = _dense_stage(x, W, a1, a2)

    hp = jnp.pad(h, ((0, NP - N), (0, 0)))
    htable = hp.reshape(NP * NCHUNK, 128)
    st1p = jnp.pad(st1, ((0, NP - N), (0, 0)))
    st2p = jnp.pad(st2, ((0, NP - N), (0, 0)))

    ei = edge_index.astype(jnp.int32)
    pad = jnp.full((EP - E,), N, jnp.int32)
    row3 = jnp.concatenate([ei[0], pad]).reshape(TILES, NBLK, BLK)
    col3 = jnp.concatenate([ei[1], pad]).reshape(TILES, NBLK, BLK)

    z128 = jnp.zeros((NP, 128), jnp.float32)

    uout, den, _ex_unused = _sc_stage(htable, st1p, st2p, row3, col3, z128)
    out = _epilogue(uout.reshape(2, NCHUNK, NP, 128),
                    den.reshape(2, NP, 128), bias)
    return out


# speedup vs baseline: 1.0425x; 1.0425x over previous
import functools

import jax
import jax.numpy as jnp
from jax import lax
from jax.experimental import pallas as pl
from jax.experimental.pallas import tpu as pltpu
from jax.experimental.pallas import tpu_sc as plsc

N = 10000
DIN = 256
DOUT = 256
H = 8
E = 160000

NP = 10112
TILES = 32
EPT = 5120
EP = TILES * EPT
BLK = 64
NBLK = EPT // BLK
STRIPE = NP // 16
NCHUNK = 16


def _dense_body(x_ref, w_ref, a1_ref, a2_ref, h_ref, st1_ref, st2_ref):
    x = x_ref[...]
    s1l, s2l = [], []
    for i in range(H):
        hi = jnp.dot(x, w_ref[i], preferred_element_type=jnp.float32)
        h_ref[:, DOUT * i:DOUT * (i + 1)] = hi
        s1l.append(jnp.dot(hi, a1_ref[i], preferred_element_type=jnp.float32))
        s2l.append(jnp.dot(hi, a2_ref[i], preferred_element_type=jnp.float32))
    zpad = jnp.zeros((x.shape[0], 128 - 2 * H), jnp.float32)
    st1_ref[...] = jnp.concatenate(s1l + s2l + [zpad], axis=1)
    st2_ref[...] = jnp.concatenate(s2l + s1l + [zpad], axis=1)


def _dense_stage(x, W, a1, a2):
    RB = 1000
    grid = (N // RB,)
    return pl.pallas_call(
        _dense_body,
        grid=grid,
        in_specs=[
            pl.BlockSpec((RB, DIN), lambda r: (r, 0)),
            pl.BlockSpec((H, DIN, DOUT), lambda r: (0, 0, 0)),
            pl.BlockSpec((H, DOUT, 1), lambda r: (0, 0, 0)),
            pl.BlockSpec((H, DOUT, 1), lambda r: (0, 0, 0)),
        ],
        out_specs=[
            pl.BlockSpec((RB, H * DOUT), lambda r: (r, 0)),
            pl.BlockSpec((RB, 128), lambda r: (r, 0)),
            pl.BlockSpec((RB, 128), lambda r: (r, 0)),
        ],
        out_shape=[
            jax.ShapeDtypeStruct((N, H * DOUT), jnp.float32),
            jax.ShapeDtypeStruct((N, 128), jnp.float32),
            jax.ShapeDtypeStruct((N, 128), jnp.float32),
        ],
    )(x, W, a1, a2)


def _sc_body(ht, s1t, s2t, rw, cl, z128, uout, dout, exout,
             row_v, col_v, g1_v, g2_v, xb_v, xb2_v, ix_v, ix2_v,
             iosem0, iosem1, ssem0, ssem1, acc):
    cid = lax.axis_index("c")
    tid = lax.axis_index("s")
    wid = tid * 2 + cid

    pltpu.sync_copy(rw.at[wid], row_v)
    pltpu.sync_copy(cl.at[wid], col_v)

    lane = lax.broadcasted_iota(jnp.int32, (16,), 0)
    m8 = lane < 8

    pltpu.sync_copy(z128.at[pl.ds(tid * STRIPE, STRIPE)],
                    acc.at[pl.ds(tid * STRIPE, STRIPE)])
    plsc.subcore_barrier()

    def blk_a(b, carry):
        pltpu.async_copy(s1t.at[row_v.at[b]], g1_v, iosem0)
        pltpu.async_copy(s2t.at[col_v.at[b]], g2_v, iosem0)
        pltpu.make_async_copy(s1t.at[row_v.at[b]], g1_v, iosem0).wait()
        pltpu.make_async_copy(s2t.at[col_v.at[b]], g2_v, iosem0).wait()

        @plsc.parallel_loop(0, BLK, 1, unroll=4)
        def edge_a(j):
            s = g1_v[j, pl.ds(0, 16)] + g2_v[j, pl.ds(0, 16)]
            e = jnp.where(s > 0.0, s, 0.2 * s)
            ex = jnp.where(m8, jnp.exp(e), 0.0)
            g2_v[j, pl.ds(0, 16)] = ex
            xb_v[pl.ds(j * 16, 16)] = ex
        pltpu.sync_copy(g2_v, acc.at[row_v.at[b]], add=True)
        pltpu.sync_copy(
            xb_v, exout.at[pl.ds((wid * EPT + b * BLK) * 16, BLK * 16)])
        return carry

    lax.fori_loop(0, NBLK, blk_a, 0)
    plsc.subcore_barrier()
    pltpu.sync_copy(acc.at[pl.ds(tid * STRIPE, STRIPE)],
                    dout.at[pl.ds(cid * NP + tid * STRIPE, STRIPE)])
    plsc.subcore_barrier()

    bufs = (g1_v, g2_v)
    xbs = (xb_v, xb2_v)
    ixs = (ix_v, ix2_v)
    iosems = (iosem0, iosem1)
    ssems = (ssem0, ssem1)

    def chunk(kk, carry):
        pltpu.sync_copy(z128.at[pl.ds(tid * STRIPE, STRIPE)],
                        acc.at[pl.ds(tid * STRIPE, STRIPE)])
        plsc.subcore_barrier()
        hd16 = jnp.full((16,), kk // 2, jnp.int32)

        def issue_in(b, p):
            for l in range(BLK // 16):
                ixs[p][pl.ds(l * 16, 16)] = (
                    col_v[b, pl.ds(l * 16, 16)] * 16 + kk)
            pltpu.async_copy(ht.at[ixs[p]], bufs[p], iosems[p])
            pltpu.async_copy(
                exout.at[pl.ds((wid * EPT + b * BLK) * 16, BLK * 16)],
                xbs[p], iosems[p])

        issue_in(0, 0)

        def pair(b2, c1):
            for p in (0, 1):
                b = b2 * 2 + p
                pltpu.make_async_copy(ht.at[ixs[p]], bufs[p],
                                      iosems[p]).wait()
                pltpu.make_async_copy(
                    exout.at[pl.ds((wid * EPT + b * BLK) * 16, BLK * 16)],
                    xbs[p], iosems[p]).wait()

                @plsc.parallel_loop(0, BLK, 1, unroll=4)
                def edge_c(j, _p=p):
                    exrow = xbs[_p][pl.ds(j * 16, 16)]
                    wv = lax.gather(
                        exrow, hd16[:, None],
                        lax.GatherDimensionNumbers(
                            offset_dims=(), collapsed_slice_dims=(0,),
                            start_index_map=(0,)),
                        slice_sizes=(1,),
                        mode=lax.GatherScatterMode.PROMISE_IN_BOUNDS)
                    for t in range(8):
                        bufs[_p][j, pl.ds(t * 16, 16)] = (
                            bufs[_p][j, pl.ds(t * 16, 16)] * wv)
                pltpu.async_copy(bufs[p], acc.at[row_v.at[b]], ssems[p],
                                 add=True)

                if p == 0:
                    @pl.when(b2 > 0)
                    def _wait_prev():
                        pltpu.make_async_copy(
                            bufs[1], acc.at[row_v.at[b - 1]],
                            ssems[1]).wait()
                    issue_in(b + 1, 1)
                else:
                    pltpu.make_async_copy(
                        bufs[0], acc.at[row_v.at[b - 1]], ssems[0]).wait()

                    @pl.when(b2 < NBLK // 2 - 1)
                    def _issue_next():
                        issue_in(b + 1, 0)
            return c1

        lax.fori_loop(0, NBLK // 2, pair, 0)
        pltpu.make_async_copy(bufs[1], acc.at[row_v.at[NBLK - 1]],
                              ssems[1]).wait()
        plsc.subcore_barrier()
        pltpu.sync_copy(
            acc.at[pl.ds(tid * STRIPE, STRIPE)],
            uout.at[pl.ds((cid * NCHUNK + kk) * NP + tid * STRIPE, STRIPE)])
        plsc.subcore_barrier()
        return carry

    lax.fori_loop(0, NCHUNK, chunk, 0)


def _sc_stage(htable, st1, st2, row3, col3, z128):
    mesh = plsc.VectorSubcoreMesh(core_axis_name="c", subcore_axis_name="s")
    kern = functools.partial(
        pl.kernel,
        out_type=[
            jax.ShapeDtypeStruct((2 * NCHUNK * NP, 128), jnp.float32),
            jax.ShapeDtypeStruct((2 * NP, 128), jnp.float32),
            jax.ShapeDtypeStruct((EP * 16,), jnp.float32),
        ],
        mesh=mesh,
        scratch_types=[
            pltpu.VMEM((NBLK, BLK), jnp.int32),
            pltpu.VMEM((NBLK, BLK), jnp.int32),
            pltpu.VMEM((BLK, 128), jnp.float32),
            pltpu.VMEM((BLK, 128), jnp.float32),
            pltpu.VMEM((BLK * 16,), jnp.float32),
            pltpu.VMEM((BLK * 16,), jnp.float32),
            pltpu.VMEM((BLK,), jnp.int32),
            pltpu.VMEM((BLK,), jnp.int32),
            pltpu.SemaphoreType.DMA,
            pltpu.SemaphoreType.DMA,
            pltpu.SemaphoreType.DMA,
            pltpu.SemaphoreType.DMA,
            pltpu.VMEM_SHARED((NP, 128), jnp.float32),
        ],
    )(_sc_body)
    return kern(htable, st1, st2, row3, col3, z128)


def _epi_body(u_ref, d_ref, b_ref, o_ref):
    dsum = d_ref[0] + d_ref[1]
    for k in range(NCHUNK):
        i, c = k // 2, k % 2
        u = u_ref[0, k] + u_ref[1, k]
        dk = dsum[:, i:i + 1]
        val = jnp.where(dk > 0.0, u / dk, 0.0)
        val = val + b_ref[i, 128 * c:128 * (c + 1)][None, :]
        o_ref[:, 128 * k:128 * (k + 1)] = jnp.maximum(val, 0.0)


def _epilogue(uout, den, bias):
    RB = 1000
    grid = (N // RB,)
    return pl.pallas_call(
        _epi_body,
        grid=grid,
        in_specs=[
            pl.BlockSpec((2, NCHUNK, RB, 128), lambda r: (0, 0, r, 0)),
            pl.BlockSpec((2, RB, 128), lambda r: (0, r, 0)),
            pl.BlockSpec((H, 2 * 128), lambda r: (0, 0)),
        ],
        out_specs=pl.BlockSpec((RB, H * DOUT), lambda r: (r, 0)),
        out_shape=jax.ShapeDtypeStruct((N, H * DOUT), jnp.float32),
    )(uout, den, bias)


def kernel(x, edge_index, W, a1, a2, bias):
    h, st1, st2 = _dense_stage(x, W, a1, a2)

    hp = jnp.pad(h, ((0, NP - N), (0, 0)))
    htable = hp.reshape(NP * NCHUNK, 128)
    st1p = jnp.pad(st1, ((0, NP - N), (0, 0)))
    st2p = jnp.pad(st2, ((0, NP - N), (0, 0)))

    ei = edge_index.astype(jnp.int32)
    pad = jnp.full((EP - E,), N, jnp.int32)
    row3 = jnp.concatenate([ei[0], pad]).reshape(TILES, NBLK, BLK)
    col3 = jnp.concatenate([ei[1], pad]).reshape(TILES, NBLK, BLK)

    z128 = jnp.zeros((NP, 128), jnp.float32)

    uout, den, _ex_unused = _sc_stage(htable, st1p, st2p, row3, col3, z128)
    out = _epilogue(uout.reshape(2, NCHUNK, NP, 128),
                    den.reshape(2, NP, 128), bias)
    return out
